# Initial kernel scaffold; baseline (speedup 1.0000x reference)
import jax
import jax.numpy as jnp
from jax.experimental import pallas as pl


def kernel(wordid, x, h, c, emb, W_iou_w, W_iou_b, U_iou_w, U_iou_b, U_f_w, U_f_b, lin_w, lin_b):
    raise NotImplementedError("write your pallas kernel here")


# speedup vs baseline: 15.7288x; 15.7288x over previous
import functools

import jax
import jax.numpy as jnp
from jax import lax
from jax.experimental import pallas as pl
from jax.experimental.pallas import tpu as pltpu
from jax.experimental.pallas import tpu_sc as plsc

_X = 128
_H = 128
_DEPTH = 15
_NL = 2 ** _DEPTH
_NN = 2 ** (_DEPTH + 1) - 1
_CHUNK = 2048
_GCH = 128


def _sc_gather(emb, wordid):
    info = plsc.get_sparse_core_info()
    ncores, nsub = info.num_cores, info.num_subcores
    nw = ncores * nsub
    bpw = _NL // nw
    nch = bpw // _GCH
    mesh = plsc.VectorSubcoreMesh(core_axis_name="c", subcore_axis_name="s")

    @functools.partial(
        pl.kernel,
        mesh=mesh,
        out_type=jax.ShapeDtypeStruct((_NL, _X), jnp.float32),
        scratch_types=[
            pltpu.VMEM((nch, _GCH), jnp.int32),
            pltpu.VMEM((2, _GCH, _X), jnp.float32),
            pltpu.SemaphoreType.DMA,
            pltpu.SemaphoreType.DMA,
        ],
    )
    def gk(emb_hbm, idx_hbm, out_hbm, idx_v, rows_v, sem0, sem1):
        wid = lax.axis_index("s") * ncores + lax.axis_index("c")
        base = wid * bpw
        for j in range(nch):
            pltpu.sync_copy(idx_hbm.at[pl.ds(base + j * _GCH, _GCH)],
                            idx_v.at[j])
        sems = (sem0, sem1)
        copies = [None] * nch
        copies[0] = pltpu.make_async_copy(
            emb_hbm.at[idx_v.at[0]], rows_v.at[0], sems[0])
        copies[0].start()
        for j in range(nch):
            if j + 1 < nch:
                copies[j + 1] = pltpu.make_async_copy(
                    emb_hbm.at[idx_v.at[j + 1]],
                    rows_v.at[(j + 1) % 2], sems[(j + 1) % 2])
                copies[j + 1].start()
            copies[j].wait()
            pltpu.sync_copy(rows_v.at[j % 2],
                            out_hbm.at[pl.ds(base + j * _GCH, _GCH)])

    return gk(emb, wordid)


def _tree_body(emb_ref, wiou_ref, biou_ref, uiou_ref, buiou_ref,
               uf_ref, bf_ref, lw_ref, lb_ref, out_ref,
               hp, cp, stage, sem):
    wiou = wiou_ref[:]
    biou = biou_ref[:]
    uiou = uiou_ref[:]
    buiou = buiou_ref[:]
    uf = uf_ref[:]
    bf = bf_ref[:]
    lw = lw_ref[:]
    lb = lb_ref[:]

    def emit(lg, m, out_row):
        stage[pl.ds(0, m), :] = lg
        copy = pltpu.make_async_copy(
            stage.at[pl.ds(0, m), :], out_ref.at[pl.ds(out_row, m), :], sem)
        copy.start()
        copy.wait()

    def act(iou, cf):
        i_ = iou[:, :_H]
        o_ = iou[:, _H:2 * _H]
        u_ = iou[:, 2 * _H:]
        c_new = jax.nn.sigmoid(i_) * jnp.tanh(u_) + cf
        h_new = jax.nn.sigmoid(o_) * jnp.tanh(c_new)
        return h_new, c_new

    def leaf_chunk(k, _):
        xc = emb_ref[pl.ds(k * _CHUNK, _CHUNK), :]
        iou = jnp.dot(xc, wiou, preferred_element_type=jnp.float32) + biou
        h_new, c_new = act(iou, 0.0)
        lg = jnp.dot(h_new, lw, preferred_element_type=jnp.float32) + lb
        emit(lg, _CHUNK, _NL - 1 + k * _CHUNK)
        hp[pl.ds(k * (_CHUNK // 2), _CHUNK // 2), :] = (
            h_new.reshape(_CHUNK // 2, 2 * _H))
        cp[pl.ds(k * (_CHUNK // 2), _CHUNK // 2), :] = (
            c_new.reshape(_CHUNK // 2, 2 * _H))
        return 0

    lax.fori_loop(0, _NL // _CHUNK, leaf_chunk, 0)

    def level_chunk(read_off, m, write_off, out_row, write_pairs):
        hp_c = hp[pl.ds(read_off, m), :]
        cp_c = cp[pl.ds(read_off, m), :]
        iou = jnp.dot(hp_c, uiou, preferred_element_type=jnp.float32) + buiou
        f = jax.nn.sigmoid(
            jnp.dot(hp_c, uf, preferred_element_type=jnp.float32) + bf)
        cf = f[:, :_H] * cp_c[:, :_H] + f[:, _H:] * cp_c[:, _H:]
        h_new, c_new = act(iou, cf)
        lg = jnp.dot(h_new, lw, preferred_element_type=jnp.float32) + lb
        emit(lg, m, out_row)
        if write_pairs:
            hp[pl.ds(write_off, m // 2), :] = h_new.reshape(m // 2, 2 * _H)
            cp[pl.ds(write_off, m // 2), :] = c_new.reshape(m // 2, 2 * _H)

    for d in range(_DEPTH - 1, -1, -1):
        n = 2 ** d
        s = n - 1
        if n > _CHUNK:
            def chunk(k, _, n=n, s=s):
                level_chunk(k * _CHUNK, _CHUNK, k * (_CHUNK // 2),
                            s + k * _CHUNK, True)
                return 0
            lax.fori_loop(0, n // _CHUNK, chunk, 0)
        else:
            level_chunk(0, n, 0, s, d > 0)


def _tree(embeds, wiou, biou, uiou, buiou, uf, bf, lw, lb):
    return pl.pallas_call(
        _tree_body,
        out_shape=jax.ShapeDtypeStruct((_NN, _H), jnp.float32),
        in_specs=[pl.BlockSpec(memory_space=pltpu.VMEM)] * 9,
        out_specs=pl.BlockSpec(memory_space=pl.ANY),
        scratch_shapes=[
            pltpu.VMEM((_NL // 2, 2 * _H), jnp.float32),
            pltpu.VMEM((_NL // 2, 2 * _H), jnp.float32),
            pltpu.VMEM((_CHUNK, _H), jnp.float32),
            pltpu.SemaphoreType.DMA,
        ],
    )(embeds, wiou, biou, uiou, buiou, uf, bf, lw, lb)


def kernel(wordid, x, h, c, emb, W_iou_w, W_iou_b, U_iou_w, U_iou_b,
           U_f_w, U_f_b, lin_w, lin_b):
    embeds = _sc_gather(emb, wordid.astype(jnp.int32))
    lw = jnp.zeros((_H, _H), jnp.float32).at[:, :5].set(lin_w)
    lb = jnp.zeros((_H,), jnp.float32).at[:5].set(lin_b)
    out = _tree(
        embeds,
        W_iou_w, W_iou_b.reshape(1, -1),
        U_iou_w, U_iou_b.reshape(1, -1),
        U_f_w, U_f_b.reshape(1, -1),
        lw, lb.reshape(1, -1),
    )
    return out[:, :5]


# speedup vs baseline: 15.8653x; 1.0087x over previous
import functools

import jax
import jax.numpy as jnp
from jax import lax
from jax.experimental import pallas as pl
from jax.experimental.pallas import tpu as pltpu
from jax.experimental.pallas import tpu_sc as plsc

_X = 128
_H = 128
_DEPTH = 15
_NL = 2 ** _DEPTH
_NN = 2 ** (_DEPTH + 1) - 1
_CHUNK = 2048
_GCH = 128


def _sc_gather(emb, wordid):
    info = plsc.get_sparse_core_info()
    ncores, nsub = info.num_cores, info.num_subcores
    nw = ncores * nsub
    bpw = _NL // nw
    nch = bpw // _GCH
    mesh = plsc.VectorSubcoreMesh(core_axis_name="c", subcore_axis_name="s")

    @functools.partial(
        pl.kernel,
        mesh=mesh,
        out_type=jax.ShapeDtypeStruct((_NL, _X), jnp.float32),
        scratch_types=[
            pltpu.VMEM((nch, _GCH), jnp.int32),
            pltpu.VMEM((2, _GCH, _X), jnp.float32),
            pltpu.SemaphoreType.DMA,
            pltpu.SemaphoreType.DMA,
        ],
    )
    def gk(emb_hbm, idx_hbm, out_hbm, idx_v, rows_v, sem0, sem1):
        wid = lax.axis_index("s") * ncores + lax.axis_index("c")
        base = wid * bpw
        for j in range(nch):
            pltpu.sync_copy(idx_hbm.at[pl.ds(base + j * _GCH, _GCH)],
                            idx_v.at[j])
        sems = (sem0, sem1)
        copies = [None] * nch
        copies[0] = pltpu.make_async_copy(
            emb_hbm.at[idx_v.at[0]], rows_v.at[0], sems[0])
        copies[0].start()
        for j in range(nch):
            if j + 1 < nch:
                copies[j + 1] = pltpu.make_async_copy(
                    emb_hbm.at[idx_v.at[j + 1]],
                    rows_v.at[(j + 1) % 2], sems[(j + 1) % 2])
                copies[j + 1].start()
            copies[j].wait()
            pltpu.sync_copy(rows_v.at[j % 2],
                            out_hbm.at[pl.ds(base + j * _GCH, _GCH)])

    return gk(emb, wordid)


def _tree_body(emb_ref, wiou_ref, biou_ref, uiou_ref, buiou_ref,
               uf_ref, bf_ref, lw_ref, lb_ref, out_ref,
               hp, cp, stage, sem):
    wiou = wiou_ref[:]
    biou = biou_ref[:]
    uiou = uiou_ref[:]
    buiou = buiou_ref[:]
    uf = uf_ref[:]
    bf = bf_ref[:]
    lw = lw_ref[:]
    lb = lb_ref[:]

    def emit(lg, m, out_row):
        stage[pl.ds(0, m), :] = lg
        copy = pltpu.make_async_copy(
            stage.at[pl.ds(0, m), :], out_ref.at[pl.ds(out_row, m), :], sem)
        copy.start()
        copy.wait()

    def act(iou, cf):
        i_ = iou[:, :_H]
        o_ = iou[:, _H:2 * _H]
        u_ = iou[:, 2 * _H:]
        c_new = jax.nn.sigmoid(i_) * jnp.tanh(u_) + cf
        h_new = jax.nn.sigmoid(o_) * jnp.tanh(c_new)
        return h_new, c_new

    def leaf_chunk(k, _):
        xc = emb_ref[pl.ds(k * _CHUNK, _CHUNK), :].astype(jnp.bfloat16)
        iou = jnp.dot(xc, wiou, preferred_element_type=jnp.float32) + biou
        h_new, c_new = act(iou, 0.0)
        hb = h_new.astype(jnp.bfloat16)
        lg = jnp.dot(hb, lw, preferred_element_type=jnp.float32) + lb
        emit(lg, _CHUNK, _NL - 1 + k * _CHUNK)
        hp[pl.ds(k * (_CHUNK // 2), _CHUNK // 2), :] = (
            hb.reshape(_CHUNK // 2, 2 * _H))
        cp[pl.ds(k * (_CHUNK // 2), _CHUNK // 2), :] = (
            c_new.reshape(_CHUNK // 2, 2 * _H))
        return 0

    lax.fori_loop(0, _NL // _CHUNK, leaf_chunk, 0)

    def level_chunk(read_off, m, write_off, out_row, write_pairs):
        hp_c = hp[pl.ds(read_off, m), :]
        cp_c = cp[pl.ds(read_off, m), :]
        iou = jnp.dot(hp_c, uiou, preferred_element_type=jnp.float32) + buiou
        f = jax.nn.sigmoid(
            jnp.dot(hp_c, uf, preferred_element_type=jnp.float32) + bf)
        cf = f[:, :_H] * cp_c[:, :_H] + f[:, _H:] * cp_c[:, _H:]
        h_new, c_new = act(iou, cf)
        hb = h_new.astype(jnp.bfloat16)
        lg = jnp.dot(hb, lw, preferred_element_type=jnp.float32) + lb
        emit(lg, m, out_row)
        if write_pairs:
            hp[pl.ds(write_off, m // 2), :] = hb.reshape(m // 2, 2 * _H)
            cp[pl.ds(write_off, m // 2), :] = c_new.reshape(m // 2, 2 * _H)

    for d in range(_DEPTH - 1, -1, -1):
        n = 2 ** d
        s = n - 1
        if n > _CHUNK:
            def chunk(k, _, n=n, s=s):
                level_chunk(k * _CHUNK, _CHUNK, k * (_CHUNK // 2),
                            s + k * _CHUNK, True)
                return 0
            lax.fori_loop(0, n // _CHUNK, chunk, 0)
        else:
            level_chunk(0, n, 0, s, d > 0)


def _tree(embeds, wiou, biou, uiou, buiou, uf, bf, lw, lb):
    return pl.pallas_call(
        _tree_body,
        out_shape=jax.ShapeDtypeStruct((_NN, _H), jnp.float32),
        in_specs=[pl.BlockSpec(memory_space=pltpu.VMEM)] * 9,
        out_specs=pl.BlockSpec(memory_space=pl.ANY),
        scratch_shapes=[
            pltpu.VMEM((_NL // 2, 2 * _H), jnp.bfloat16),
            pltpu.VMEM((_NL // 2, 2 * _H), jnp.float32),
            pltpu.VMEM((_CHUNK, _H), jnp.float32),
            pltpu.SemaphoreType.DMA,
        ],
    )(embeds, wiou, biou, uiou, buiou, uf, bf, lw, lb)


def kernel(wordid, x, h, c, emb, W_iou_w, W_iou_b, U_iou_w, U_iou_b,
           U_f_w, U_f_b, lin_w, lin_b):
    embeds = _sc_gather(emb, wordid.astype(jnp.int32))
    lw = jnp.zeros((_H, _H), jnp.float32).at[:, :5].set(lin_w)
    lb = jnp.zeros((_H,), jnp.float32).at[:5].set(lin_b)
    out = _tree(
        embeds,
        W_iou_w.astype(jnp.bfloat16), W_iou_b.reshape(1, -1),
        U_iou_w.astype(jnp.bfloat16), U_iou_b.reshape(1, -1),
        U_f_w.astype(jnp.bfloat16), U_f_b.reshape(1, -1),
        lw.astype(jnp.bfloat16), lb.reshape(1, -1),
    )
    return out[:, :5]


# speedup vs baseline: 21.5936x; 1.3611x over previous
import functools

import jax
import jax.numpy as jnp
from jax import lax
from jax.experimental import pallas as pl
from jax.experimental.pallas import tpu as pltpu
from jax.experimental.pallas import tpu_sc as plsc

_X = 128
_H = 128
_DEPTH = 15
_NL = 2 ** _DEPTH
_NN = 2 ** (_DEPTH + 1) - 1
_CHUNK = 2048
_GCH = 128


def _sc_gather(emb, wordid):
    info = plsc.get_sparse_core_info()
    ncores, nsub = info.num_cores, info.num_subcores
    nw = ncores * nsub
    bpw = _NL // nw
    nch = bpw // _GCH
    mesh = plsc.VectorSubcoreMesh(core_axis_name="c", subcore_axis_name="s")

    @functools.partial(
        pl.kernel,
        mesh=mesh,
        out_type=jax.ShapeDtypeStruct((_NL, _X), jnp.float32),
        scratch_types=[
            pltpu.VMEM((nch, _GCH), jnp.int32),
            pltpu.VMEM((2, _GCH, _X), jnp.float32),
            pltpu.SemaphoreType.DMA,
            pltpu.SemaphoreType.DMA,
        ],
    )
    def gk(emb_hbm, idx_hbm, out_hbm, idx_v, rows_v, sem0, sem1):
        wid = lax.axis_index("s") * ncores + lax.axis_index("c")
        base = wid * bpw
        for j in range(nch):
            pltpu.sync_copy(idx_hbm.at[pl.ds(base + j * _GCH, _GCH)],
                            idx_v.at[j])
        sems = (sem0, sem1)
        copies = [None] * nch
        copies[0] = pltpu.make_async_copy(
            emb_hbm.at[idx_v.at[0]], rows_v.at[0], sems[0])
        copies[0].start()
        for j in range(nch):
            if j + 1 < nch:
                copies[j + 1] = pltpu.make_async_copy(
                    emb_hbm.at[idx_v.at[j + 1]],
                    rows_v.at[(j + 1) % 2], sems[(j + 1) % 2])
                copies[j + 1].start()
            copies[j].wait()
            pltpu.sync_copy(rows_v.at[j % 2],
                            out_hbm.at[pl.ds(base + j * _GCH, _GCH)])

    return gk(emb, wordid)


def _sigmoid(x):
    return 0.5 * jnp.tanh(0.5 * x) + 0.5


def _tree_body(emb_ref, wiou_ref, biou_ref, ucat_ref, bcat_ref,
               lw_ref, lb_ref, out_ref,
               hp, cp, stage, sstage, sems, ssems):
    wiou = wiou_ref[:]
    biou = biou_ref[:]
    ucat = ucat_ref[:]
    bcat = bcat_ref[:]
    lw = lw_ref[:]
    lb = lb_ref[:]

    def emit_loop(lg, k, out_row):
        slot = lax.rem(k, 2)

        @pl.when(k >= 2)
        def _():
            pltpu.make_async_copy(
                stage.at[slot], out_ref.at[pl.ds(out_row, _CHUNK), :],
                sems.at[slot]).wait()

        stage[slot] = lg
        pltpu.make_async_copy(
            stage.at[slot], out_ref.at[pl.ds(out_row, _CHUNK), :],
            sems.at[slot]).start()

    def drain(trips, rows_fn):
        for k in range(max(trips - 2, 0), trips):
            pltpu.make_async_copy(
                stage.at[k % 2], out_ref.at[pl.ds(rows_fn(k), _CHUNK), :],
                sems.at[k % 2]).wait()

    def act(iou, cf):
        i_ = iou[:, :_H]
        o_ = iou[:, _H:2 * _H]
        u_ = iou[:, 2 * _H:3 * _H]
        c_new = _sigmoid(i_) * jnp.tanh(u_) + cf
        h_new = _sigmoid(o_) * jnp.tanh(c_new)
        return h_new, c_new

    def leaf_chunk(k, _):
        xc = emb_ref[pl.ds(k * _CHUNK, _CHUNK), :].astype(jnp.bfloat16)
        iou = jnp.dot(xc, wiou, preferred_element_type=jnp.float32) + biou
        h_new, c_new = act(iou, 0.0)
        hb = h_new.astype(jnp.bfloat16)
        lg = jnp.dot(hb, lw, preferred_element_type=jnp.float32) + lb
        emit_loop(lg, k, _NL - 1 + k * _CHUNK)
        hp[pl.ds(k * (_CHUNK // 2), _CHUNK // 2), :] = (
            hb.reshape(_CHUNK // 2, 2 * _H))
        cp[pl.ds(k * (_CHUNK // 2), _CHUNK // 2), :] = (
            c_new.reshape(_CHUNK // 2, 2 * _H))
        return 0

    lax.fori_loop(0, _NL // _CHUNK, leaf_chunk, 0)
    drain(_NL // _CHUNK, lambda k: _NL - 1 + k * _CHUNK)

    def level_math(hp_c, cp_c):
        iouf = jnp.dot(hp_c, ucat, preferred_element_type=jnp.float32) + bcat
        f_l = _sigmoid(iouf[:, 3 * _H:4 * _H])
        f_r = _sigmoid(iouf[:, 4 * _H:])
        cf = f_l * cp_c[:, :_H] + f_r * cp_c[:, _H:]
        h_new, c_new = act(iouf, cf)
        hb = h_new.astype(jnp.bfloat16)
        lg = jnp.dot(hb, lw, preferred_element_type=jnp.float32) + lb
        return hb, c_new, lg

    for d in range(_DEPTH - 1, -1, -1):
        n = 2 ** d
        s = n - 1
        if n >= _CHUNK:
            def chunk(k, _, s=s):
                hb, c_new, lg = level_math(hp[pl.ds(k * _CHUNK, _CHUNK), :],
                                           cp[pl.ds(k * _CHUNK, _CHUNK), :])
                emit_loop(lg, k, s + k * _CHUNK)
                hp[pl.ds(k * (_CHUNK // 2), _CHUNK // 2), :] = (
                    hb.reshape(_CHUNK // 2, 2 * _H))
                cp[pl.ds(k * (_CHUNK // 2), _CHUNK // 2), :] = (
                    c_new.reshape(_CHUNK // 2, 2 * _H))
                return 0
            lax.fori_loop(0, n // _CHUNK, chunk, 0)
            drain(n // _CHUNK, lambda k, s=s: s + k * _CHUNK)

    soff = 0
    pending = []
    for d in range(10, -1, -1):
        n = 2 ** d
        s = n - 1
        hb, c_new, lg = level_math(hp[pl.ds(0, n), :], cp[pl.ds(0, n), :])
        sstage[pl.ds(soff, n), :] = lg
        copy = pltpu.make_async_copy(
            sstage.at[pl.ds(soff, n), :], out_ref.at[pl.ds(s, n), :],
            ssems.at[10 - d])
        copy.start()
        pending.append(copy)
        soff += n
        if d > 0:
            hp[pl.ds(0, n // 2), :] = hb.reshape(n // 2, 2 * _H)
            cp[pl.ds(0, n // 2), :] = c_new.reshape(n // 2, 2 * _H)
    for copy in pending:
        copy.wait()


def _tree(embeds, wiou, biou, ucat, bcat, lw, lb):
    return pl.pallas_call(
        _tree_body,
        out_shape=jax.ShapeDtypeStruct((_NN, _H), jnp.float32),
        in_specs=[pl.BlockSpec(memory_space=pltpu.VMEM)] * 7,
        out_specs=pl.BlockSpec(memory_space=pl.ANY),
        scratch_shapes=[
            pltpu.VMEM((_NL // 2, 2 * _H), jnp.bfloat16),
            pltpu.VMEM((_NL // 2, 2 * _H), jnp.float32),
            pltpu.VMEM((2, _CHUNK, _H), jnp.float32),
            pltpu.VMEM((_CHUNK, _H), jnp.float32),
            pltpu.SemaphoreType.DMA((2,)),
            pltpu.SemaphoreType.DMA((11,)),
        ],
    )(embeds, wiou, biou, ucat, bcat, lw, lb)


def kernel(wordid, x, h, c, emb, W_iou_w, W_iou_b, U_iou_w, U_iou_b,
           U_f_w, U_f_b, lin_w, lin_b):
    embeds = _sc_gather(emb, wordid.astype(jnp.int32))
    lw = jnp.zeros((_H, _H), jnp.float32).at[:, :5].set(lin_w)
    lb = jnp.zeros((_H,), jnp.float32).at[:5].set(lin_b)
    ucat = jnp.concatenate([U_iou_w, U_f_w], axis=1)
    bcat = jnp.concatenate([U_iou_b, U_f_b])
    out = _tree(
        embeds,
        W_iou_w.astype(jnp.bfloat16), W_iou_b.reshape(1, -1),
        ucat.astype(jnp.bfloat16), bcat.reshape(1, -1),
        lw.astype(jnp.bfloat16), lb.reshape(1, -1),
    )
    return out[:, :5]


# speedup vs baseline: 21.7403x; 1.0068x over previous
import functools

import jax
import jax.numpy as jnp
from jax import lax
from jax.experimental import pallas as pl
from jax.experimental.pallas import tpu as pltpu
from jax.experimental.pallas import tpu_sc as plsc

_X = 128
_H = 128
_DEPTH = 15
_NL = 2 ** _DEPTH
_NN = 2 ** (_DEPTH + 1) - 1
_CHUNK = 2048
_GCH = 128
_GBUF = 6


def _sc_gather(emb, wordid):
    info = plsc.get_sparse_core_info()
    ncores, nsub = info.num_cores, info.num_subcores
    nw = ncores * nsub
    bpw = _NL // nw
    nch = bpw // _GCH
    mesh = plsc.VectorSubcoreMesh(core_axis_name="c", subcore_axis_name="s")

    @functools.partial(
        pl.kernel,
        mesh=mesh,
        out_type=jax.ShapeDtypeStruct((_NL, _X), jnp.float32),
        scratch_types=[
            pltpu.VMEM((nch, _GCH), jnp.int32),
            pltpu.VMEM((_GBUF, _GCH, _X), jnp.float32),
            pltpu.SemaphoreType.DMA((_GBUF,)),
            pltpu.SemaphoreType.DMA((_GBUF,)),
        ],
    )
    def gk(emb_hbm, idx_hbm, out_hbm, idx_v, rows_v, gsem, wsem):
        wid = lax.axis_index("s") * ncores + lax.axis_index("c")
        base = wid * bpw
        for j in range(nch):
            pltpu.sync_copy(idx_hbm.at[pl.ds(base + j * _GCH, _GCH)],
                            idx_v.at[j])

        def gather(j):
            return pltpu.make_async_copy(
                emb_hbm.at[idx_v.at[j]], rows_v.at[j % _GBUF],
                gsem.at[j % _GBUF])

        def write(j):
            return pltpu.make_async_copy(
                rows_v.at[j % _GBUF],
                out_hbm.at[pl.ds(base + j * _GCH, _GCH)],
                wsem.at[j % _GBUF])

        for j in range(min(_GBUF, nch)):
            gather(j).start()
        writes = []
        for j in range(nch):
            gather(j).wait()
            w = write(j)
            w.start()
            writes.append(w)
            if j + _GBUF < nch:
                writes.remove(w)
                w.wait()
                gather(j + _GBUF).start()
        for w in writes:
            w.wait()

    return gk(emb, wordid)


def _sigmoid(x):
    return 0.5 * jnp.tanh(0.5 * x) + 0.5


def _tree_body(emb_hbm, wiou_ref, biou_ref, ucat_ref, bcat_ref,
               lw_ref, lb_ref, out_ref,
               hp, cp, xbuf, stage, sstage, xsems, sems, ssems):
    wiou = wiou_ref[:]
    biou = biou_ref[:]
    ucat = ucat_ref[:]
    bcat = bcat_ref[:]
    lw = lw_ref[:]
    lb = lb_ref[:]

    def head(hb, m):
        return jnp.dot(hb, lw, preferred_element_type=jnp.float32) + lb

    def emit_loop(lg, k, out_row):
        slot = lax.rem(k, 2)

        @pl.when(k >= 2)
        def _():
            pltpu.make_async_copy(
                stage.at[slot], out_ref.at[pl.ds(out_row, _CHUNK), :],
                sems.at[slot]).wait()

        stage[slot] = lg
        pltpu.make_async_copy(
            stage.at[slot], out_ref.at[pl.ds(out_row, _CHUNK), :],
            sems.at[slot]).start()

    def drain(trips, rows_fn):
        for k in range(max(trips - 2, 0), trips):
            pltpu.make_async_copy(
                stage.at[k % 2], out_ref.at[pl.ds(rows_fn(k), _CHUNK), :],
                sems.at[k % 2]).wait()

    def act(iou, cf):
        i_ = iou[:, :_H]
        o_ = iou[:, _H:2 * _H]
        u_ = iou[:, 2 * _H:3 * _H]
        c_new = _sigmoid(i_) * jnp.tanh(u_) + cf
        h_new = _sigmoid(o_) * jnp.tanh(c_new)
        return h_new, c_new

    def xcopy(k):
        return pltpu.make_async_copy(
            emb_hbm.at[pl.ds(k * _CHUNK, _CHUNK), :],
            xbuf.at[lax.rem(k, 2)], xsems.at[lax.rem(k, 2)])

    ltrips = _NL // _CHUNK
    xcopy(0).start()

    def leaf_chunk(k, _):
        xcopy(k).wait()

        @pl.when(k + 1 < ltrips)
        def _():
            xcopy(k + 1).start()

        xc = xbuf[lax.rem(k, 2)].astype(jnp.bfloat16)
        iou = jnp.dot(xc, wiou, preferred_element_type=jnp.float32) + biou
        h_new, c_new = act(iou, 0.0)
        hb = h_new.astype(jnp.bfloat16)
        emit_loop(head(hb, _CHUNK), k, _NL - 1 + k * _CHUNK)
        hp[pl.ds(k * (_CHUNK // 2), _CHUNK // 2), :] = (
            hb.reshape(_CHUNK // 2, 2 * _H))
        cp[pl.ds(k * (_CHUNK // 2), _CHUNK // 2), :] = (
            c_new.reshape(_CHUNK // 2, 2 * _H))
        return 0

    lax.fori_loop(0, ltrips, leaf_chunk, 0)
    drain(ltrips, lambda k: _NL - 1 + k * _CHUNK)

    def level_math(hp_c, cp_c, m):
        iouf = jnp.dot(hp_c, ucat, preferred_element_type=jnp.float32) + bcat
        f_l = _sigmoid(iouf[:, 3 * _H:4 * _H])
        f_r = _sigmoid(iouf[:, 4 * _H:])
        cf = f_l * cp_c[:, :_H] + f_r * cp_c[:, _H:]
        h_new, c_new = act(iouf, cf)
        hb = h_new.astype(jnp.bfloat16)
        return hb, c_new, head(hb, m)

    for d in range(_DEPTH - 1, -1, -1):
        n = 2 ** d
        s = n - 1
        if n >= _CHUNK:
            def chunk(k, _, s=s):
                hb, c_new, lgt = level_math(
                    hp[pl.ds(k * _CHUNK, _CHUNK), :],
                    cp[pl.ds(k * _CHUNK, _CHUNK), :], _CHUNK)
                emit_loop(lgt, k, s + k * _CHUNK)
                hp[pl.ds(k * (_CHUNK // 2), _CHUNK // 2), :] = (
                    hb.reshape(_CHUNK // 2, 2 * _H))
                cp[pl.ds(k * (_CHUNK // 2), _CHUNK // 2), :] = (
                    c_new.reshape(_CHUNK // 2, 2 * _H))
                return 0
            lax.fori_loop(0, n // _CHUNK, chunk, 0)
            drain(n // _CHUNK, lambda k, s=s: s + k * _CHUNK)

    soff = 0
    pending = []
    for d in range(10, -1, -1):
        n = 2 ** d
        s = n - 1
        hb, c_new, lg = level_math(hp[pl.ds(0, n), :], cp[pl.ds(0, n), :], n)
        sstage[pl.ds(soff, n), :] = lg
        copy = pltpu.make_async_copy(
            sstage.at[pl.ds(soff, n), :], out_ref.at[pl.ds(s, n), :],
            ssems.at[10 - d])
        copy.start()
        pending.append(copy)
        soff += n
        if d > 0:
            hp[pl.ds(0, n // 2), :] = hb.reshape(n // 2, 2 * _H)
            cp[pl.ds(0, n // 2), :] = c_new.reshape(n // 2, 2 * _H)
    for copy in pending:
        copy.wait()


def _tree(embeds, wiou, biou, ucat, bcat, lw, lb):
    return pl.pallas_call(
        _tree_body,
        out_shape=jax.ShapeDtypeStruct((_NN, _H), jnp.float32),
        in_specs=[pl.BlockSpec(memory_space=pl.ANY)]
        + [pl.BlockSpec(memory_space=pltpu.VMEM)] * 6,
        out_specs=pl.BlockSpec(memory_space=pl.ANY),
        scratch_shapes=[
            pltpu.VMEM((_NL // 2, 2 * _H), jnp.bfloat16),
            pltpu.VMEM((_NL // 2, 2 * _H), jnp.float32),
            pltpu.VMEM((2, _CHUNK, _X), jnp.float32),
            pltpu.VMEM((2, _CHUNK, _H), jnp.float32),
            pltpu.VMEM((_CHUNK, _H), jnp.float32),
            pltpu.SemaphoreType.DMA((2,)),
            pltpu.SemaphoreType.DMA((2,)),
            pltpu.SemaphoreType.DMA((11,)),
        ],
    )(embeds, wiou, biou, ucat, bcat, lw, lb)


def kernel(wordid, x, h, c, emb, W_iou_w, W_iou_b, U_iou_w, U_iou_b,
           U_f_w, U_f_b, lin_w, lin_b):
    embeds = _sc_gather(emb, wordid.astype(jnp.int32))
    lw = jnp.zeros((_H, _H), jnp.float32).at[:, :5].set(lin_w)
    lb = jnp.zeros((_H,), jnp.float32).at[:5].set(lin_b)
    ucat = jnp.concatenate([U_iou_w, U_f_w], axis=1)
    bcat = jnp.concatenate([U_iou_b, U_f_b])
    out = _tree(
        embeds,
        W_iou_w.astype(jnp.bfloat16), W_iou_b.reshape(1, -1),
        ucat.astype(jnp.bfloat16), bcat.reshape(1, -1),
        lw.astype(jnp.bfloat16), lb.reshape(1, -1),
    )
    return out[:, :5]


# speedup vs baseline: 21.8557x; 1.0053x over previous
import functools

import jax
import jax.numpy as jnp
from jax import lax
from jax.experimental import pallas as pl
from jax.experimental.pallas import tpu as pltpu
from jax.experimental.pallas import tpu_sc as plsc

_X = 128
_H = 128
_DEPTH = 15
_NL = 2 ** _DEPTH
_NN = 2 ** (_DEPTH + 1) - 1
_CHUNK = 2048
_GCH = 128
_GBUF = 6


def _sc_gather(emb, wordid):
    info = plsc.get_sparse_core_info()
    ncores, nsub = info.num_cores, info.num_subcores
    nw = ncores * nsub
    bpw = _NL // nw
    nch = bpw // _GCH
    mesh = plsc.VectorSubcoreMesh(core_axis_name="c", subcore_axis_name="s")

    @functools.partial(
        pl.kernel,
        mesh=mesh,
        out_type=jax.ShapeDtypeStruct((_NL, _X), jnp.float32),
        scratch_types=[
            pltpu.VMEM((nch, _GCH), jnp.int32),
            pltpu.VMEM((_GBUF, _GCH, _X), jnp.float32),
            pltpu.SemaphoreType.DMA((_GBUF,)),
            pltpu.SemaphoreType.DMA((_GBUF,)),
        ],
    )
    def gk(emb_hbm, idx_hbm, out_hbm, idx_v, rows_v, gsem, wsem):
        wid = lax.axis_index("s") * ncores + lax.axis_index("c")
        base = wid * bpw
        for j in range(nch):
            pltpu.sync_copy(idx_hbm.at[pl.ds(base + j * _GCH, _GCH)],
                            idx_v.at[j])

        def gather(j):
            return pltpu.make_async_copy(
                emb_hbm.at[idx_v.at[j]], rows_v.at[j % _GBUF],
                gsem.at[j % _GBUF])

        def write(j):
            return pltpu.make_async_copy(
                rows_v.at[j % _GBUF],
                out_hbm.at[pl.ds(base + j * _GCH, _GCH)],
                wsem.at[j % _GBUF])

        for j in range(min(_GBUF, nch)):
            gather(j).start()
        writes = []
        for j in range(nch):
            gather(j).wait()
            w = write(j)
            w.start()
            writes.append(w)
            if j + _GBUF < nch:
                writes.remove(w)
                w.wait()
                gather(j + _GBUF).start()
        for w in writes:
            w.wait()

    return gk(emb, wordid)


def _sigmoid(x):
    return 0.5 * jnp.tanh(0.5 * x) + 0.5


def _tree_body(emb_hbm, wiou_ref, biou_ref, uiou_ref, uf_ref,
               biouu_ref, bfu_ref, lin_ref, linb_ref, out_ref,
               hp, cp, xbuf, stage, sstage, xsems, sems, ssems):
    wiou = wiou_ref[:].astype(jnp.bfloat16)
    biou = biou_ref[:]
    ucat = jnp.concatenate([uiou_ref[:].astype(jnp.bfloat16),
                            uf_ref[:].astype(jnp.bfloat16)], axis=1)
    bcat = jnp.concatenate([biouu_ref[:], bfu_ref[:]], axis=1)
    lw = jnp.concatenate(
        [lin_ref[:].astype(jnp.bfloat16),
         jnp.zeros((_H, _H - 5), jnp.bfloat16)], axis=1)
    lb = jnp.concatenate(
        [linb_ref[:], jnp.zeros((1, _H - 5), jnp.float32)], axis=1)

    def head(hb, m):
        return jnp.dot(hb, lw, preferred_element_type=jnp.float32) + lb

    def emit_loop(lg, k, out_row):
        slot = lax.rem(k, 2)

        @pl.when(k >= 2)
        def _():
            pltpu.make_async_copy(
                stage.at[slot], out_ref.at[pl.ds(out_row, _CHUNK), :],
                sems.at[slot]).wait()

        stage[slot] = lg
        pltpu.make_async_copy(
            stage.at[slot], out_ref.at[pl.ds(out_row, _CHUNK), :],
            sems.at[slot]).start()

    def drain(trips, rows_fn):
        for k in range(max(trips - 2, 0), trips):
            pltpu.make_async_copy(
                stage.at[k % 2], out_ref.at[pl.ds(rows_fn(k), _CHUNK), :],
                sems.at[k % 2]).wait()

    def act(iou, cf):
        i_ = iou[:, :_H]
        o_ = iou[:, _H:2 * _H]
        u_ = iou[:, 2 * _H:3 * _H]
        c_new = _sigmoid(i_) * jnp.tanh(u_) + cf
        h_new = _sigmoid(o_) * jnp.tanh(c_new)
        return h_new, c_new

    def xcopy(k):
        return pltpu.make_async_copy(
            emb_hbm.at[pl.ds(k * _CHUNK, _CHUNK), :],
            xbuf.at[lax.rem(k, 2)], xsems.at[lax.rem(k, 2)])

    ltrips = _NL // _CHUNK
    xcopy(0).start()

    def leaf_chunk(k, _):
        xcopy(k).wait()

        @pl.when(k + 1 < ltrips)
        def _():
            xcopy(k + 1).start()

        xc = xbuf[lax.rem(k, 2)].astype(jnp.bfloat16)
        iou = jnp.dot(xc, wiou, preferred_element_type=jnp.float32) + biou
        h_new, c_new = act(iou, 0.0)
        hb = h_new.astype(jnp.bfloat16)
        emit_loop(head(hb, _CHUNK), k, _NL - 1 + k * _CHUNK)
        hp[pl.ds(k * (_CHUNK // 2), _CHUNK // 2), :] = (
            hb.reshape(_CHUNK // 2, 2 * _H))
        cp[pl.ds(k * (_CHUNK // 2), _CHUNK // 2), :] = (
            c_new.reshape(_CHUNK // 2, 2 * _H))
        return 0

    lax.fori_loop(0, ltrips, leaf_chunk, 0)
    drain(ltrips, lambda k: _NL - 1 + k * _CHUNK)

    def level_math(hp_c, cp_c, m):
        iouf = jnp.dot(hp_c, ucat, preferred_element_type=jnp.float32) + bcat
        f_l = _sigmoid(iouf[:, 3 * _H:4 * _H])
        f_r = _sigmoid(iouf[:, 4 * _H:])
        cf = f_l * cp_c[:, :_H] + f_r * cp_c[:, _H:]
        h_new, c_new = act(iouf, cf)
        hb = h_new.astype(jnp.bfloat16)
        return hb, c_new, head(hb, m)

    for d in range(_DEPTH - 1, -1, -1):
        n = 2 ** d
        s = n - 1
        if n >= _CHUNK:
            def chunk(k, _, s=s):
                hb, c_new, lgt = level_math(
                    hp[pl.ds(k * _CHUNK, _CHUNK), :],
                    cp[pl.ds(k * _CHUNK, _CHUNK), :], _CHUNK)
                emit_loop(lgt, k, s + k * _CHUNK)
                hp[pl.ds(k * (_CHUNK // 2), _CHUNK // 2), :] = (
                    hb.reshape(_CHUNK // 2, 2 * _H))
                cp[pl.ds(k * (_CHUNK // 2), _CHUNK // 2), :] = (
                    c_new.reshape(_CHUNK // 2, 2 * _H))
                return 0
            lax.fori_loop(0, n // _CHUNK, chunk, 0)
            drain(n // _CHUNK, lambda k, s=s: s + k * _CHUNK)

    soff = 0
    pending = []
    for d in range(10, -1, -1):
        n = 2 ** d
        s = n - 1
        hb, c_new, lg = level_math(hp[pl.ds(0, n), :], cp[pl.ds(0, n), :], n)
        sstage[pl.ds(soff, n), :] = lg
        copy = pltpu.make_async_copy(
            sstage.at[pl.ds(soff, n), :], out_ref.at[pl.ds(s, n), :],
            ssems.at[10 - d])
        copy.start()
        pending.append(copy)
        soff += n
        if d > 0:
            hp[pl.ds(0, n // 2), :] = hb.reshape(n // 2, 2 * _H)
            cp[pl.ds(0, n // 2), :] = c_new.reshape(n // 2, 2 * _H)
    for copy in pending:
        copy.wait()


def _tree(embeds, *weights):
    return pl.pallas_call(
        _tree_body,
        out_shape=jax.ShapeDtypeStruct((_NN, _H), jnp.float32),
        in_specs=[pl.BlockSpec(memory_space=pl.ANY)]
        + [pl.BlockSpec(memory_space=pltpu.VMEM)] * 8,
        out_specs=pl.BlockSpec(memory_space=pl.ANY),
        scratch_shapes=[
            pltpu.VMEM((_NL // 2, 2 * _H), jnp.bfloat16),
            pltpu.VMEM((_NL // 2, 2 * _H), jnp.float32),
            pltpu.VMEM((2, _CHUNK, _X), jnp.float32),
            pltpu.VMEM((2, _CHUNK, _H), jnp.float32),
            pltpu.VMEM((_CHUNK, _H), jnp.float32),
            pltpu.SemaphoreType.DMA((2,)),
            pltpu.SemaphoreType.DMA((2,)),
            pltpu.SemaphoreType.DMA((11,)),
        ],
    )(embeds, *weights)


def kernel(wordid, x, h, c, emb, W_iou_w, W_iou_b, U_iou_w, U_iou_b,
           U_f_w, U_f_b, lin_w, lin_b):
    embeds = _sc_gather(emb, wordid.astype(jnp.int32))
    out = _tree(
        embeds,
        W_iou_w, W_iou_b.reshape(1, -1),
        U_iou_w, U_f_w,
        U_iou_b.reshape(1, -1), U_f_b.reshape(1, -1),
        lin_w, lin_b.reshape(1, -1),
    )
    return out[:, :5]


# speedup vs baseline: 24.4636x; 1.1193x over previous
import functools

import jax
import jax.numpy as jnp
from jax import lax
from jax.experimental import pallas as pl
from jax.experimental.pallas import tpu as pltpu
from jax.experimental.pallas import tpu_sc as plsc

_X = 128
_H = 128
_DEPTH = 15
_NL = 2 ** _DEPTH
_NN = 2 ** (_DEPTH + 1) - 1
_CHUNK = 2048
_GCH = 128
_GBUF = 6


def _sc_gather(emb, wordid):
    info = plsc.get_sparse_core_info()
    ncores, nsub = info.num_cores, info.num_subcores
    nw = ncores * nsub
    bpw = _NL // nw
    nch = bpw // _GCH
    mesh = plsc.VectorSubcoreMesh(core_axis_name="c", subcore_axis_name="s")

    @functools.partial(
        pl.kernel,
        mesh=mesh,
        out_type=jax.ShapeDtypeStruct((_NL, _X), jnp.float32),
        scratch_types=[
            pltpu.VMEM((nch, _GCH), jnp.int32),
            pltpu.VMEM((_GBUF, _GCH, _X), jnp.float32),
            pltpu.SemaphoreType.DMA((_GBUF,)),
            pltpu.SemaphoreType.DMA((_GBUF,)),
        ],
    )
    def gk(emb_hbm, idx_hbm, out_hbm, idx_v, rows_v, gsem, wsem):
        wid = lax.axis_index("s") * ncores + lax.axis_index("c")
        base = wid * bpw
        for j in range(nch):
            pltpu.sync_copy(idx_hbm.at[pl.ds(base + j * _GCH, _GCH)],
                            idx_v.at[j])

        def gather(j):
            return pltpu.make_async_copy(
                emb_hbm.at[idx_v.at[j]], rows_v.at[j % _GBUF],
                gsem.at[j % _GBUF])

        def write(j):
            return pltpu.make_async_copy(
                rows_v.at[j % _GBUF],
                out_hbm.at[pl.ds(base + j * _GCH, _GCH)],
                wsem.at[j % _GBUF])

        for j in range(min(_GBUF, nch)):
            gather(j).start()
        writes = []
        for j in range(nch):
            gather(j).wait()
            w = write(j)
            w.start()
            writes.append(w)
            if j + _GBUF < nch:
                writes.remove(w)
                w.wait()
                gather(j + _GBUF).start()
        for w in writes:
            w.wait()

    return gk(emb, wordid)


def _sigmoid(x):
    return 0.5 * jnp.tanh(0.5 * x) + 0.5


_COLOFF = {}
_c = 0
for _lv in [15] + list(range(14, -1, -1)):
    _COLOFF[_lv] = _c
    _c += max(2 ** _lv, 128)
_NCOL = _c


def _tree_body(emb_hbm, wiou_ref, biou_ref, uiou_ref, uf_ref,
               biouu_ref, bfu_ref, lint_ref, lbt_ref, out_ref,
               hp, cp, xbuf, stage, sstage, xsems, sems, ssems):
    wiou = wiou_ref[:].astype(jnp.bfloat16)
    biou = biou_ref[:]
    ucat = jnp.concatenate([uiou_ref[:].astype(jnp.bfloat16),
                            uf_ref[:].astype(jnp.bfloat16)], axis=1)
    bcat = jnp.concatenate([biouu_ref[:], bfu_ref[:]], axis=1)
    lwt = lint_ref[:]
    lwt_bf = lwt.astype(jnp.bfloat16)
    lbt = lbt_ref[:]

    def head(hmat, m):
        a = lwt_bf if hmat.dtype == jnp.bfloat16 else lwt
        lgt = lax.dot_general(a, hmat, (((1,), (1,)), ((), ())),
                              preferred_element_type=jnp.float32)
        return lgt + (lbt if m >= _CHUNK else lbt[:, :m])

    def emit_loop(lgt, k, out_col):
        slot = lax.rem(k, 2)

        @pl.when(k >= 2)
        def _():
            pltpu.make_async_copy(
                stage.at[slot], out_ref.at[:, pl.ds(out_col, _CHUNK)],
                sems.at[slot]).wait()

        stage[slot] = lgt
        pltpu.make_async_copy(
            stage.at[slot], out_ref.at[:, pl.ds(out_col, _CHUNK)],
            sems.at[slot]).start()

    def drain(trips, cols_fn):
        for k in range(max(trips - 2, 0), trips):
            pltpu.make_async_copy(
                stage.at[k % 2], out_ref.at[:, pl.ds(cols_fn(k), _CHUNK)],
                sems.at[k % 2]).wait()

    def act(iou, cf):
        i_ = iou[:, :_H]
        o_ = iou[:, _H:2 * _H]
        u_ = iou[:, 2 * _H:3 * _H]
        c_new = _sigmoid(i_) * jnp.tanh(u_) + cf
        h_new = _sigmoid(o_) * jnp.tanh(c_new)
        return h_new, c_new

    def xcopy(k):
        return pltpu.make_async_copy(
            emb_hbm.at[pl.ds(k * _CHUNK, _CHUNK), :],
            xbuf.at[lax.rem(k, 2)], xsems.at[lax.rem(k, 2)])

    ltrips = _NL // _CHUNK
    xcopy(0).start()

    def leaf_chunk(k, _):
        xcopy(k).wait()

        @pl.when(k + 1 < ltrips)
        def _():
            xcopy(k + 1).start()

        xc = xbuf[lax.rem(k, 2)].astype(jnp.bfloat16)
        iou = jnp.dot(xc, wiou, preferred_element_type=jnp.float32) + biou
        h_new, c_new = act(iou, 0.0)
        hb = h_new.astype(jnp.bfloat16)
        emit_loop(head(hb, _CHUNK), k, _COLOFF[15] + k * _CHUNK)
        hp[pl.ds(k * (_CHUNK // 2), _CHUNK // 2), :] = (
            hb.reshape(_CHUNK // 2, 2 * _H))
        cp[pl.ds(k * (_CHUNK // 2), _CHUNK // 2), :] = (
            c_new.reshape(_CHUNK // 2, 2 * _H))
        return 0

    lax.fori_loop(0, ltrips, leaf_chunk, 0)
    drain(ltrips, lambda k: _COLOFF[15] + k * _CHUNK)

    def level_math(hp_c, cp_c, m):
        iouf = jnp.dot(hp_c, ucat, preferred_element_type=jnp.float32) + bcat
        f_l = _sigmoid(iouf[:, 3 * _H:4 * _H])
        f_r = _sigmoid(iouf[:, 4 * _H:])
        cf = f_l * cp_c[:, :_H] + f_r * cp_c[:, _H:]
        h_new, c_new = act(iouf, cf)
        hb = h_new.astype(jnp.bfloat16)
        if m >= _CHUNK:
            lgt = head(hb, m)
        else:
            hmat = h_new
            if m < 128:
                hmat = jnp.concatenate(
                    [h_new, jnp.zeros((128 - m, _H), jnp.float32)], axis=0)
            lgt = head(hmat, max(m, 128))
        return hb, c_new, lgt

    for d in range(_DEPTH - 1, -1, -1):
        n = 2 ** d
        if n >= _CHUNK:
            def chunk(k, _, d=d):
                hb, c_new, lgt = level_math(
                    hp[pl.ds(k * _CHUNK, _CHUNK), :],
                    cp[pl.ds(k * _CHUNK, _CHUNK), :], _CHUNK)
                emit_loop(lgt, k, _COLOFF[d] + k * _CHUNK)
                hp[pl.ds(k * (_CHUNK // 2), _CHUNK // 2), :] = (
                    hb.reshape(_CHUNK // 2, 2 * _H))
                cp[pl.ds(k * (_CHUNK // 2), _CHUNK // 2), :] = (
                    c_new.reshape(_CHUNK // 2, 2 * _H))
                return 0
            lax.fori_loop(0, n // _CHUNK, chunk, 0)
            drain(n // _CHUNK, lambda k, d=d: _COLOFF[d] + k * _CHUNK)

    pending = []
    for d in range(10, -1, -1):
        n = 2 ** d
        w = max(n, 128)
        i = 10 - d
        hb, c_new, lgt = level_math(hp[pl.ds(0, n), :], cp[pl.ds(0, n), :], n)
        sstage[pl.ds(8 * i, 5), pl.ds(0, w)] = lgt
        copy = pltpu.make_async_copy(
            sstage.at[pl.ds(8 * i, 5), pl.ds(0, w)],
            out_ref.at[:, pl.ds(_COLOFF[d], w)], ssems.at[i])
        copy.start()
        pending.append(copy)
        if d > 0:
            hp[pl.ds(0, n // 2), :] = hb.reshape(n // 2, 2 * _H)
            cp[pl.ds(0, n // 2), :] = c_new.reshape(n // 2, 2 * _H)
    for copy in pending:
        copy.wait()


def _tree(embeds, *weights):
    return pl.pallas_call(
        _tree_body,
        out_shape=jax.ShapeDtypeStruct((5, _NCOL), jnp.float32),
        in_specs=[pl.BlockSpec(memory_space=pl.ANY)]
        + [pl.BlockSpec(memory_space=pltpu.VMEM)] * 8,
        out_specs=pl.BlockSpec(memory_space=pl.ANY),
        scratch_shapes=[
            pltpu.VMEM((_NL // 2, 2 * _H), jnp.bfloat16),
            pltpu.VMEM((_NL // 2, 2 * _H), jnp.float32),
            pltpu.VMEM((2, _CHUNK, _X), jnp.float32),
            pltpu.VMEM((2, 5, _CHUNK), jnp.float32),
            pltpu.VMEM((88, _CHUNK), jnp.float32),
            pltpu.SemaphoreType.DMA((2,)),
            pltpu.SemaphoreType.DMA((2,)),
            pltpu.SemaphoreType.DMA((11,)),
        ],
    )(embeds, *weights)


def kernel(wordid, x, h, c, emb, W_iou_w, W_iou_b, U_iou_w, U_iou_b,
           U_f_w, U_f_b, lin_w, lin_b):
    embeds = _sc_gather(emb, wordid.astype(jnp.int32))
    out = _tree(
        embeds,
        W_iou_w, W_iou_b.reshape(1, -1),
        U_iou_w, U_f_w,
        U_iou_b.reshape(1, -1), U_f_b.reshape(1, -1),
        lin_w.T, jnp.broadcast_to(lin_b.reshape(5, 1), (5, _CHUNK)),
    )
    parts = [lax.slice(out, (0, _COLOFF[d]), (5, _COLOFF[d] + 2 ** d))
             for d in range(15)]
    parts.append(lax.slice(out, (0, _COLOFF[15]), (5, _COLOFF[15] + _NL)))
    return jnp.concatenate(parts, axis=1).T


# speedup vs baseline: 25.6456x; 1.0483x over previous
import functools

import jax
import jax.numpy as jnp
from jax import lax
from jax.experimental import pallas as pl
from jax.experimental.pallas import tpu as pltpu
from jax.experimental.pallas import tpu_sc as plsc

_X = 128
_H = 128
_DEPTH = 15
_NL = 2 ** _DEPTH
_NN = 2 ** (_DEPTH + 1) - 1
_CHUNK = 2048
_GCH = 128
_GBUF = 6


def _sc_gather(emb, wordid):
    info = plsc.get_sparse_core_info()
    ncores, nsub = info.num_cores, info.num_subcores
    nw = ncores * nsub
    bpw = _NL // nw
    nch = bpw // _GCH
    mesh = plsc.VectorSubcoreMesh(core_axis_name="c", subcore_axis_name="s")

    @functools.partial(
        pl.kernel,
        mesh=mesh,
        out_type=jax.ShapeDtypeStruct((_NL, _X), jnp.float32),
        scratch_types=[
            pltpu.VMEM((nch, _GCH), jnp.int32),
            pltpu.VMEM((_GBUF, _GCH, _X), jnp.float32),
            pltpu.SemaphoreType.DMA((_GBUF,)),
            pltpu.SemaphoreType.DMA((_GBUF,)),
        ],
    )
    def gk(emb_hbm, idx_hbm, out_hbm, idx_v, rows_v, gsem, wsem):
        wid = lax.axis_index("s") * ncores + lax.axis_index("c")
        base = wid * bpw
        pltpu.sync_copy(idx_hbm.at[pl.ds(wid * nch, nch)], idx_v)

        def gather(j):
            return pltpu.make_async_copy(
                emb_hbm.at[idx_v.at[j]], rows_v.at[j % _GBUF],
                gsem.at[j % _GBUF])

        def write(j):
            return pltpu.make_async_copy(
                rows_v.at[j % _GBUF],
                out_hbm.at[pl.ds(base + j * _GCH, _GCH)],
                wsem.at[j % _GBUF])

        for j in range(min(_GBUF, nch)):
            gather(j).start()
        writes = []
        for j in range(nch):
            gather(j).wait()
            w = write(j)
            w.start()
            writes.append(w)
            if j + _GBUF < nch:
                writes.remove(w)
                w.wait()
                gather(j + _GBUF).start()
        for w in writes:
            w.wait()

    return gk(emb, wordid.reshape(_NL // _GCH, _GCH))


def _sigmoid(x):
    return 0.5 * jnp.tanh(x) + 0.5


_COLOFF = {}
_c = 0
for _lv in [15] + list(range(14, -1, -1)):
    _COLOFF[_lv] = _c
    _c += max(2 ** _lv, 128)
_NCOL = _c


def _tree_body(emb_hbm, wiou_ref, biou_ref, uiou_ref, uf_ref,
               biouu_ref, bfu_ref, lint_ref, lbt_ref, out_ref,
               hp, cp, xbuf, stage, sstage, xsems, sems, ssems):
    wi = wiou_ref[:]
    wiou = jnp.concatenate([wi[:, :2 * _H] * 0.5,
                            wi[:, 2 * _H:]], axis=1).astype(jnp.bfloat16)
    bi = biou_ref[:]
    biou = jnp.concatenate([bi[:, :2 * _H] * 0.5, bi[:, 2 * _H:]], axis=1)
    ui = uiou_ref[:]
    ucat = jnp.concatenate([ui[:, :2 * _H] * 0.5, ui[:, 2 * _H:],
                            uf_ref[:] * 0.5], axis=1).astype(jnp.bfloat16)
    ub = biouu_ref[:]
    bcat = jnp.concatenate([ub[:, :2 * _H] * 0.5, ub[:, 2 * _H:],
                            bfu_ref[:] * 0.5], axis=1)
    lwt = lint_ref[:]
    lwt_bf = lwt.astype(jnp.bfloat16)
    lbt = lbt_ref[:]

    def head(hmat, m):
        a = lwt_bf if hmat.dtype == jnp.bfloat16 else lwt
        lgt = lax.dot_general(a, hmat, (((1,), (1,)), ((), ())),
                              preferred_element_type=jnp.float32)
        return lgt + (lbt if m >= _CHUNK else lbt[:, :m])

    def emit_loop(lgt, k, out_col):
        slot = lax.rem(k, 2)

        @pl.when(k >= 2)
        def _():
            pltpu.make_async_copy(
                stage.at[slot], out_ref.at[:, pl.ds(out_col, _CHUNK)],
                sems.at[slot]).wait()

        stage[slot] = lgt
        pltpu.make_async_copy(
            stage.at[slot], out_ref.at[:, pl.ds(out_col, _CHUNK)],
            sems.at[slot]).start()

    def drain(trips, cols_fn):
        for k in range(max(trips - 2, 0), trips):
            pltpu.make_async_copy(
                stage.at[k % 2], out_ref.at[:, pl.ds(cols_fn(k), _CHUNK)],
                sems.at[k % 2]).wait()

    def act(iou, cf):
        i_ = iou[:, :_H]
        o_ = iou[:, _H:2 * _H]
        u_ = iou[:, 2 * _H:3 * _H]
        c_new = _sigmoid(i_) * jnp.tanh(u_) + cf
        h_new = _sigmoid(o_) * jnp.tanh(c_new)
        return h_new, c_new

    def xcopy(k):
        return pltpu.make_async_copy(
            emb_hbm.at[pl.ds(k * _CHUNK, _CHUNK), :],
            xbuf.at[lax.rem(k, 2)], xsems.at[lax.rem(k, 2)])

    ltrips = _NL // _CHUNK
    xcopy(0).start()

    def leaf_chunk(k, _):
        xcopy(k).wait()

        @pl.when(k + 1 < ltrips)
        def _():
            xcopy(k + 1).start()

        xc = xbuf[lax.rem(k, 2)].astype(jnp.bfloat16)
        iou = jnp.dot(xc, wiou, preferred_element_type=jnp.float32) + biou
        h_new, c_new = act(iou, 0.0)
        hb = h_new.astype(jnp.bfloat16)
        emit_loop(head(hb, _CHUNK), k, _COLOFF[15] + k * _CHUNK)
        hp[pl.ds(k * (_CHUNK // 2), _CHUNK // 2), :] = (
            hb.reshape(_CHUNK // 2, 2 * _H))
        cp[pl.ds(k * (_CHUNK // 2), _CHUNK // 2), :] = (
            c_new.reshape(_CHUNK // 2, 2 * _H))
        return 0

    lax.fori_loop(0, ltrips, leaf_chunk, 0)
    drain(ltrips, lambda k: _COLOFF[15] + k * _CHUNK)

    def level_math(hp_c, cp_c, m):
        iouf = jnp.dot(hp_c, ucat, preferred_element_type=jnp.float32) + bcat
        f_l = _sigmoid(iouf[:, 3 * _H:4 * _H])
        f_r = _sigmoid(iouf[:, 4 * _H:])
        cf = f_l * cp_c[:, :_H] + f_r * cp_c[:, _H:]
        h_new, c_new = act(iouf, cf)
        hb = h_new.astype(jnp.bfloat16)
        if m >= _CHUNK:
            lgt = head(hb, m)
        else:
            hmat = h_new
            if m < 128:
                hmat = jnp.concatenate(
                    [h_new, jnp.zeros((128 - m, _H), jnp.float32)], axis=0)
            lgt = head(hmat, max(m, 128))
        return hb, c_new, lgt

    for d in range(_DEPTH - 1, -1, -1):
        n = 2 ** d
        if n >= _CHUNK:
            def chunk(k, _, d=d):
                hb, c_new, lgt = level_math(
                    hp[pl.ds(k * _CHUNK, _CHUNK), :],
                    cp[pl.ds(k * _CHUNK, _CHUNK), :], _CHUNK)
                emit_loop(lgt, k, _COLOFF[d] + k * _CHUNK)
                hp[pl.ds(k * (_CHUNK // 2), _CHUNK // 2), :] = (
                    hb.reshape(_CHUNK // 2, 2 * _H))
                cp[pl.ds(k * (_CHUNK // 2), _CHUNK // 2), :] = (
                    c_new.reshape(_CHUNK // 2, 2 * _H))
                return 0
            lax.fori_loop(0, n // _CHUNK, chunk, 0)
            drain(n // _CHUNK, lambda k, d=d: _COLOFF[d] + k * _CHUNK)

    pending = []
    for d in range(10, -1, -1):
        n = 2 ** d
        w = max(n, 128)
        i = 10 - d
        hb, c_new, lgt = level_math(hp[pl.ds(0, n), :], cp[pl.ds(0, n), :], n)
        sstage[pl.ds(8 * i, 5), pl.ds(0, w)] = lgt
        copy = pltpu.make_async_copy(
            sstage.at[pl.ds(8 * i, 5), pl.ds(0, w)],
            out_ref.at[:, pl.ds(_COLOFF[d], w)], ssems.at[i])
        copy.start()
        pending.append(copy)
        if d > 0:
            hp[pl.ds(0, n // 2), :] = hb.reshape(n // 2, 2 * _H)
            cp[pl.ds(0, n // 2), :] = c_new.reshape(n // 2, 2 * _H)
    for copy in pending:
        copy.wait()


def _tree(embeds, *weights):
    return pl.pallas_call(
        _tree_body,
        out_shape=jax.ShapeDtypeStruct((5, _NCOL), jnp.float32),
        in_specs=[pl.BlockSpec(memory_space=pl.ANY)]
        + [pl.BlockSpec(memory_space=pltpu.VMEM)] * 8,
        out_specs=pl.BlockSpec(memory_space=pl.ANY),
        scratch_shapes=[
            pltpu.VMEM((_NL // 2, 2 * _H), jnp.bfloat16),
            pltpu.VMEM((_NL // 2, 2 * _H), jnp.float32),
            pltpu.VMEM((2, _CHUNK, _X), jnp.float32),
            pltpu.VMEM((2, 5, _CHUNK), jnp.float32),
            pltpu.VMEM((88, _CHUNK), jnp.float32),
            pltpu.SemaphoreType.DMA((2,)),
            pltpu.SemaphoreType.DMA((2,)),
            pltpu.SemaphoreType.DMA((11,)),
        ],
    )(embeds, *weights)


def kernel(wordid, x, h, c, emb, W_iou_w, W_iou_b, U_iou_w, U_iou_b,
           U_f_w, U_f_b, lin_w, lin_b):
    embeds = _sc_gather(emb, wordid.astype(jnp.int32))
    out = _tree(
        embeds,
        W_iou_w, W_iou_b.reshape(1, -1),
        U_iou_w, U_f_w,
        U_iou_b.reshape(1, -1), U_f_b.reshape(1, -1),
        lin_w.T, jnp.broadcast_to(lin_b.reshape(5, 1), (5, _CHUNK)),
    )
    parts = [lax.slice(out, (0, _COLOFF[d]), (5, _COLOFF[d] + 2 ** d))
             for d in range(15)]
    parts.append(lax.slice(out, (0, _COLOFF[15]), (5, _COLOFF[15] + _NL)))
    return jnp.concatenate(parts, axis=1).T


# speedup vs baseline: 28.0317x; 1.0930x over previous
import functools

import jax
import jax.numpy as jnp
from jax import lax
from jax.experimental import pallas as pl
from jax.experimental.pallas import tpu as pltpu
from jax.experimental.pallas import tpu_sc as plsc

_X = 128
_H = 128
_DEPTH = 15
_NL = 2 ** _DEPTH
_NN = 2 ** (_DEPTH + 1) - 1
_CHUNK = 4096
_SMALL_TOP = _CHUNK.bit_length() - 2
_NSMALL = _SMALL_TOP + 1
_GCH = 128
_GBUF = 6


def _sc_gather(emb, wordid):
    info = plsc.get_sparse_core_info()
    ncores, nsub = info.num_cores, info.num_subcores
    nw = ncores * nsub
    bpw = _NL // nw
    nch = bpw // _GCH
    mesh = plsc.VectorSubcoreMesh(core_axis_name="c", subcore_axis_name="s")

    @functools.partial(
        pl.kernel,
        mesh=mesh,
        out_type=jax.ShapeDtypeStruct((_NL, _X), jnp.float32),
        scratch_types=[
            pltpu.VMEM((nch, _GCH), jnp.int32),
            pltpu.VMEM((_GBUF, _GCH, _X), jnp.float32),
            pltpu.SemaphoreType.DMA((_GBUF,)),
            pltpu.SemaphoreType.DMA((_GBUF,)),
        ],
    )
    def gk(emb_hbm, idx_hbm, out_hbm, idx_v, rows_v, gsem, wsem):
        wid = lax.axis_index("s") * ncores + lax.axis_index("c")
        base = wid * bpw
        pltpu.sync_copy(idx_hbm.at[pl.ds(wid * nch, nch)], idx_v)

        def gather(j):
            return pltpu.make_async_copy(
                emb_hbm.at[idx_v.at[j]], rows_v.at[j % _GBUF],
                gsem.at[j % _GBUF])

        def write(j):
            return pltpu.make_async_copy(
                rows_v.at[j % _GBUF],
                out_hbm.at[pl.ds(base + j * _GCH, _GCH)],
                wsem.at[j % _GBUF])

        for j in range(min(_GBUF, nch)):
            gather(j).start()
        writes = []
        for j in range(nch):
            gather(j).wait()
            w = write(j)
            w.start()
            writes.append(w)
            if j + _GBUF < nch:
                writes.remove(w)
                w.wait()
                gather(j + _GBUF).start()
        for w in writes:
            w.wait()

    return gk(emb, wordid.reshape(_NL // _GCH, _GCH))


def _sigmoid(x):
    return 0.5 * jnp.tanh(x) + 0.5


_COLOFF = {}
_c = 0
for _lv in [15] + list(range(14, -1, -1)):
    _COLOFF[_lv] = _c
    _c += max(2 ** _lv, 128)
_NCOL = _c


def _tree_body(emb_hbm, wiou_ref, biou_ref, uiou_ref, uf_ref,
               biouu_ref, bfu_ref, lint_ref, lbt_ref, out_ref,
               hp, cp, xbuf, stage, sstage, xsems, sems, ssems):
    wi = wiou_ref[:]
    wiou = jnp.concatenate([wi[:, :2 * _H] * 0.5,
                            wi[:, 2 * _H:]], axis=1).astype(jnp.bfloat16)
    bi = biou_ref[:]
    biou = jnp.concatenate([bi[:, :2 * _H] * 0.5, bi[:, 2 * _H:]], axis=1)
    ui = uiou_ref[:]
    ucat = jnp.concatenate([ui[:, :2 * _H] * 0.5, ui[:, 2 * _H:],
                            uf_ref[:] * 0.5], axis=1).astype(jnp.bfloat16)
    ub = biouu_ref[:]
    bcat = jnp.concatenate([ub[:, :2 * _H] * 0.5, ub[:, 2 * _H:],
                            bfu_ref[:] * 0.5], axis=1)
    lwt = lint_ref[:]
    lwt_bf = lwt.astype(jnp.bfloat16)
    lbt = lbt_ref[:]

    def head(hmat, m):
        a = lwt_bf if hmat.dtype == jnp.bfloat16 else lwt
        lgt = lax.dot_general(a, hmat, (((1,), (1,)), ((), ())),
                              preferred_element_type=jnp.float32)
        return lgt + (lbt if m >= _CHUNK else lbt[:, :m])

    def emit_loop(lgt, k, out_col):
        slot = lax.rem(k, 2)

        @pl.when(k >= 2)
        def _():
            pltpu.make_async_copy(
                stage.at[slot], out_ref.at[:, pl.ds(out_col, _CHUNK)],
                sems.at[slot]).wait()

        stage[slot] = lgt
        pltpu.make_async_copy(
            stage.at[slot], out_ref.at[:, pl.ds(out_col, _CHUNK)],
            sems.at[slot]).start()

    def drain(trips, cols_fn):
        for k in range(max(trips - 2, 0), trips):
            pltpu.make_async_copy(
                stage.at[k % 2], out_ref.at[:, pl.ds(cols_fn(k), _CHUNK)],
                sems.at[k % 2]).wait()

    def act(iou, cf):
        i_ = iou[:, :_H]
        o_ = iou[:, _H:2 * _H]
        u_ = iou[:, 2 * _H:3 * _H]
        c_new = _sigmoid(i_) * jnp.tanh(u_) + cf
        h_new = _sigmoid(o_) * jnp.tanh(c_new)
        return h_new, c_new

    def xcopy(k):
        return pltpu.make_async_copy(
            emb_hbm.at[pl.ds(k * _CHUNK, _CHUNK), :],
            xbuf.at[lax.rem(k, 2)], xsems.at[lax.rem(k, 2)])

    ltrips = _NL // _CHUNK
    xcopy(0).start()

    def leaf_chunk(k, _):
        xcopy(k).wait()

        @pl.when(k + 1 < ltrips)
        def _():
            xcopy(k + 1).start()

        xc = xbuf[lax.rem(k, 2)].astype(jnp.bfloat16)
        iou = jnp.dot(xc, wiou, preferred_element_type=jnp.float32) + biou
        h_new, c_new = act(iou, 0.0)
        hb = h_new.astype(jnp.bfloat16)
        emit_loop(head(hb, _CHUNK), k, _COLOFF[15] + k * _CHUNK)
        hp[pl.ds(k * (_CHUNK // 2), _CHUNK // 2), :] = (
            hb.reshape(_CHUNK // 2, 2 * _H))
        cp[pl.ds(k * (_CHUNK // 2), _CHUNK // 2), :] = (
            c_new.reshape(_CHUNK // 2, 2 * _H))
        return 0

    lax.fori_loop(0, ltrips, leaf_chunk, 0)
    drain(ltrips, lambda k: _COLOFF[15] + k * _CHUNK)

    def level_math(hp_c, cp_c, m):
        iouf = jnp.dot(hp_c, ucat, preferred_element_type=jnp.float32) + bcat
        f_l = _sigmoid(iouf[:, 3 * _H:4 * _H])
        f_r = _sigmoid(iouf[:, 4 * _H:])
        cf = f_l * cp_c[:, :_H] + f_r * cp_c[:, _H:]
        h_new, c_new = act(iouf, cf)
        hb = h_new.astype(jnp.bfloat16)
        if m >= _CHUNK:
            lgt = head(hb, m)
        else:
            hmat = h_new
            if m < 128:
                hmat = jnp.concatenate(
                    [h_new, jnp.zeros((128 - m, _H), jnp.float32)], axis=0)
            lgt = head(hmat, max(m, 128))
        return hb, c_new, lgt

    for d in range(_DEPTH - 1, -1, -1):
        n = 2 ** d
        if n >= _CHUNK:
            def chunk(k, _, d=d):
                hb, c_new, lgt = level_math(
                    hp[pl.ds(k * _CHUNK, _CHUNK), :],
                    cp[pl.ds(k * _CHUNK, _CHUNK), :], _CHUNK)
                emit_loop(lgt, k, _COLOFF[d] + k * _CHUNK)
                hp[pl.ds(k * (_CHUNK // 2), _CHUNK // 2), :] = (
                    hb.reshape(_CHUNK // 2, 2 * _H))
                cp[pl.ds(k * (_CHUNK // 2), _CHUNK // 2), :] = (
                    c_new.reshape(_CHUNK // 2, 2 * _H))
                return 0
            lax.fori_loop(0, n // _CHUNK, chunk, 0)
            drain(n // _CHUNK, lambda k, d=d: _COLOFF[d] + k * _CHUNK)

    pending = []
    for d in range(_SMALL_TOP, -1, -1):
        n = 2 ** d
        w = max(n, 128)
        i = _SMALL_TOP - d
        hb, c_new, lgt = level_math(hp[pl.ds(0, n), :], cp[pl.ds(0, n), :], n)
        sstage[pl.ds(8 * i, 5), pl.ds(0, w)] = lgt
        copy = pltpu.make_async_copy(
            sstage.at[pl.ds(8 * i, 5), pl.ds(0, w)],
            out_ref.at[:, pl.ds(_COLOFF[d], w)], ssems.at[i])
        copy.start()
        pending.append(copy)
        if d > 0:
            hp[pl.ds(0, n // 2), :] = hb.reshape(n // 2, 2 * _H)
            cp[pl.ds(0, n // 2), :] = c_new.reshape(n // 2, 2 * _H)
    for copy in pending:
        copy.wait()


def _tree(embeds, *weights):
    return pl.pallas_call(
        _tree_body,
        out_shape=jax.ShapeDtypeStruct((5, _NCOL), jnp.float32),
        in_specs=[pl.BlockSpec(memory_space=pl.ANY)]
        + [pl.BlockSpec(memory_space=pltpu.VMEM)] * 8,
        out_specs=pl.BlockSpec(memory_space=pl.ANY),
        scratch_shapes=[
            pltpu.VMEM((_NL // 2, 2 * _H), jnp.bfloat16),
            pltpu.VMEM((_NL // 2, 2 * _H), jnp.float32),
            pltpu.VMEM((2, _CHUNK, _X), jnp.float32),
            pltpu.VMEM((2, 5, _CHUNK), jnp.float32),
            pltpu.VMEM((8 * _NSMALL, _CHUNK), jnp.float32),
            pltpu.SemaphoreType.DMA((2,)),
            pltpu.SemaphoreType.DMA((2,)),
            pltpu.SemaphoreType.DMA((_NSMALL,)),
        ],
    )(embeds, *weights)


def kernel(wordid, x, h, c, emb, W_iou_w, W_iou_b, U_iou_w, U_iou_b,
           U_f_w, U_f_b, lin_w, lin_b):
    embeds = _sc_gather(emb, wordid.astype(jnp.int32))
    out = _tree(
        embeds,
        W_iou_w, W_iou_b.reshape(1, -1),
        U_iou_w, U_f_w,
        U_iou_b.reshape(1, -1), U_f_b.reshape(1, -1),
        lin_w.T, jnp.broadcast_to(lin_b.reshape(5, 1), (5, _CHUNK)),
    )
    parts = [lax.slice(out, (0, _COLOFF[d]), (5, _COLOFF[d] + 2 ** d))
             for d in range(15)]
    parts.append(lax.slice(out, (0, _COLOFF[15]), (5, _COLOFF[15] + _NL)))
    return jnp.concatenate(parts, axis=1).T


# speedup vs baseline: 28.8167x; 1.0280x over previous
import functools

import jax
import jax.numpy as jnp
from jax import lax
from jax.experimental import pallas as pl
from jax.experimental.pallas import tpu as pltpu
from jax.experimental.pallas import tpu_sc as plsc

_X = 128
_H = 128
_DEPTH = 15
_NL = 2 ** _DEPTH
_NN = 2 ** (_DEPTH + 1) - 1
_CHUNK = 8192
_SMALL_TOP = _CHUNK.bit_length() - 2
_NSMALL = _SMALL_TOP + 1
_GCH = 128
_GBUF = 6


def _sc_gather(emb, wordid):
    info = plsc.get_sparse_core_info()
    ncores, nsub = info.num_cores, info.num_subcores
    nw = ncores * nsub
    bpw = _NL // nw
    nch = bpw // _GCH
    mesh = plsc.VectorSubcoreMesh(core_axis_name="c", subcore_axis_name="s")

    @functools.partial(
        pl.kernel,
        mesh=mesh,
        out_type=jax.ShapeDtypeStruct((_NL, _X), jnp.float32),
        scratch_types=[
            pltpu.VMEM((nch, _GCH), jnp.int32),
            pltpu.VMEM((_GBUF, _GCH, _X), jnp.float32),
            pltpu.SemaphoreType.DMA((_GBUF,)),
            pltpu.SemaphoreType.DMA((_GBUF,)),
        ],
    )
    def gk(emb_hbm, idx_hbm, out_hbm, idx_v, rows_v, gsem, wsem):
        wid = lax.axis_index("s") * ncores + lax.axis_index("c")
        base = wid * bpw
        pltpu.sync_copy(idx_hbm.at[pl.ds(wid * nch, nch)], idx_v)

        def gather(j):
            return pltpu.make_async_copy(
                emb_hbm.at[idx_v.at[j]], rows_v.at[j % _GBUF],
                gsem.at[j % _GBUF])

        def write(j):
            return pltpu.make_async_copy(
                rows_v.at[j % _GBUF],
                out_hbm.at[pl.ds(base + j * _GCH, _GCH)],
                wsem.at[j % _GBUF])

        for j in range(min(_GBUF, nch)):
            gather(j).start()
        writes = []
        for j in range(nch):
            gather(j).wait()
            w = write(j)
            w.start()
            writes.append(w)
            if j + _GBUF < nch:
                writes.remove(w)
                w.wait()
                gather(j + _GBUF).start()
        for w in writes:
            w.wait()

    return gk(emb, wordid.reshape(_NL // _GCH, _GCH))


def _sigmoid(x):
    return 0.5 * jnp.tanh(x) + 0.5


_COLOFF = {}
_c = 0
for _lv in [15] + list(range(14, -1, -1)):
    _COLOFF[_lv] = _c
    _c += max(2 ** _lv, 128)
_NCOL = _c


def _tree_body(emb_hbm, wiou_ref, biou_ref, uiou_ref, uf_ref,
               biouu_ref, bfu_ref, lint_ref, lbt_ref, out_ref,
               hp, cp, xbuf, stage, sstage, xsems, sems, ssems):
    wi = wiou_ref[:]
    wiou = jnp.concatenate([wi[:, :2 * _H] * 0.5,
                            wi[:, 2 * _H:]], axis=1).astype(jnp.bfloat16)
    bi = biou_ref[:]
    biou = jnp.concatenate([bi[:, :2 * _H] * 0.5, bi[:, 2 * _H:]], axis=1)
    ui = uiou_ref[:]
    ucat = jnp.concatenate([ui[:, :2 * _H] * 0.5, ui[:, 2 * _H:],
                            uf_ref[:] * 0.5], axis=1).astype(jnp.bfloat16)
    ub = biouu_ref[:]
    bcat = jnp.concatenate([ub[:, :2 * _H] * 0.5, ub[:, 2 * _H:],
                            bfu_ref[:] * 0.5], axis=1)
    lwt = lint_ref[:]
    lwt_bf = lwt.astype(jnp.bfloat16)
    lbt = lbt_ref[:]

    def head(hmat, m):
        a = lwt_bf if hmat.dtype == jnp.bfloat16 else lwt
        lgt = lax.dot_general(a, hmat, (((1,), (1,)), ((), ())),
                              preferred_element_type=jnp.float32)
        return lgt + (lbt if m >= _CHUNK else lbt[:, :m])

    def emit_loop(lgt, k, out_col):
        slot = lax.rem(k, 2)

        @pl.when(k >= 2)
        def _():
            pltpu.make_async_copy(
                stage.at[slot], out_ref.at[:, pl.ds(out_col, _CHUNK)],
                sems.at[slot]).wait()

        stage[slot] = lgt
        pltpu.make_async_copy(
            stage.at[slot], out_ref.at[:, pl.ds(out_col, _CHUNK)],
            sems.at[slot]).start()

    def drain(trips, cols_fn):
        for k in range(max(trips - 2, 0), trips):
            pltpu.make_async_copy(
                stage.at[k % 2], out_ref.at[:, pl.ds(cols_fn(k), _CHUNK)],
                sems.at[k % 2]).wait()

    def act(iou, cf):
        i_ = iou[:, :_H]
        o_ = iou[:, _H:2 * _H]
        u_ = iou[:, 2 * _H:3 * _H]
        c_new = _sigmoid(i_) * jnp.tanh(u_) + cf
        h_new = _sigmoid(o_) * jnp.tanh(c_new)
        return h_new, c_new

    def xcopy(k):
        return pltpu.make_async_copy(
            emb_hbm.at[pl.ds(k * _CHUNK, _CHUNK), :],
            xbuf.at[lax.rem(k, 2)], xsems.at[lax.rem(k, 2)])

    ltrips = _NL // _CHUNK
    xcopy(0).start()

    def leaf_chunk(k, _):
        xcopy(k).wait()

        @pl.when(k + 1 < ltrips)
        def _():
            xcopy(k + 1).start()

        xc = xbuf[lax.rem(k, 2)].astype(jnp.bfloat16)
        iou = jnp.dot(xc, wiou, preferred_element_type=jnp.float32) + biou
        h_new, c_new = act(iou, 0.0)
        hb = h_new.astype(jnp.bfloat16)
        emit_loop(head(hb, _CHUNK), k, _COLOFF[15] + k * _CHUNK)
        hp[pl.ds(k * (_CHUNK // 2), _CHUNK // 2), :] = (
            hb.reshape(_CHUNK // 2, 2 * _H))
        cp[pl.ds(k * (_CHUNK // 2), _CHUNK // 2), :] = (
            c_new.reshape(_CHUNK // 2, 2 * _H))
        return 0

    lax.fori_loop(0, ltrips, leaf_chunk, 0)
    drain(ltrips, lambda k: _COLOFF[15] + k * _CHUNK)

    def level_math(hp_c, cp_c, m):
        iouf = jnp.dot(hp_c, ucat, preferred_element_type=jnp.float32) + bcat
        f_l = _sigmoid(iouf[:, 3 * _H:4 * _H])
        f_r = _sigmoid(iouf[:, 4 * _H:])
        cf = f_l * cp_c[:, :_H] + f_r * cp_c[:, _H:]
        h_new, c_new = act(iouf, cf)
        hb = h_new.astype(jnp.bfloat16)
        if m >= _CHUNK:
            lgt = head(hb, m)
        else:
            hmat = h_new
            if m < 128:
                hmat = jnp.concatenate(
                    [h_new, jnp.zeros((128 - m, _H), jnp.float32)], axis=0)
            lgt = head(hmat, max(m, 128))
        return hb, c_new, lgt

    for d in range(_DEPTH - 1, -1, -1):
        n = 2 ** d
        if n >= _CHUNK:
            def chunk(k, _, d=d):
                hb, c_new, lgt = level_math(
                    hp[pl.ds(k * _CHUNK, _CHUNK), :],
                    cp[pl.ds(k * _CHUNK, _CHUNK), :], _CHUNK)
                emit_loop(lgt, k, _COLOFF[d] + k * _CHUNK)
                hp[pl.ds(k * (_CHUNK // 2), _CHUNK // 2), :] = (
                    hb.reshape(_CHUNK // 2, 2 * _H))
                cp[pl.ds(k * (_CHUNK // 2), _CHUNK // 2), :] = (
                    c_new.reshape(_CHUNK // 2, 2 * _H))
                return 0
            lax.fori_loop(0, n // _CHUNK, chunk, 0)
            drain(n // _CHUNK, lambda k, d=d: _COLOFF[d] + k * _CHUNK)

    pending = []
    for d in range(_SMALL_TOP, -1, -1):
        n = 2 ** d
        w = max(n, 128)
        i = _SMALL_TOP - d
        hb, c_new, lgt = level_math(hp[pl.ds(0, n), :], cp[pl.ds(0, n), :], n)
        sstage[pl.ds(8 * i, 5), pl.ds(0, w)] = lgt
        copy = pltpu.make_async_copy(
            sstage.at[pl.ds(8 * i, 5), pl.ds(0, w)],
            out_ref.at[:, pl.ds(_COLOFF[d], w)], ssems.at[i])
        copy.start()
        pending.append(copy)
        if d > 0:
            hp[pl.ds(0, n // 2), :] = hb.reshape(n // 2, 2 * _H)
            cp[pl.ds(0, n // 2), :] = c_new.reshape(n // 2, 2 * _H)
    for copy in pending:
        copy.wait()


def _tree(embeds, *weights):
    return pl.pallas_call(
        _tree_body,
        out_shape=jax.ShapeDtypeStruct((5, _NCOL), jnp.float32),
        in_specs=[pl.BlockSpec(memory_space=pl.ANY)]
        + [pl.BlockSpec(memory_space=pltpu.VMEM)] * 8,
        out_specs=pl.BlockSpec(memory_space=pl.ANY),
        scratch_shapes=[
            pltpu.VMEM((_NL // 2, 2 * _H), jnp.bfloat16),
            pltpu.VMEM((_NL // 2, 2 * _H), jnp.float32),
            pltpu.VMEM((2, _CHUNK, _X), jnp.float32),
            pltpu.VMEM((2, 5, _CHUNK), jnp.float32),
            pltpu.VMEM((8 * _NSMALL, _CHUNK // 2), jnp.float32),
            pltpu.SemaphoreType.DMA((2,)),
            pltpu.SemaphoreType.DMA((2,)),
            pltpu.SemaphoreType.DMA((_NSMALL,)),
        ],
    )(embeds, *weights)


def kernel(wordid, x, h, c, emb, W_iou_w, W_iou_b, U_iou_w, U_iou_b,
           U_f_w, U_f_b, lin_w, lin_b):
    embeds = _sc_gather(emb, wordid.astype(jnp.int32))
    out = _tree(
        embeds,
        W_iou_w, W_iou_b.reshape(1, -1),
        U_iou_w, U_f_w,
        U_iou_b.reshape(1, -1), U_f_b.reshape(1, -1),
        lin_w.T, jnp.broadcast_to(lin_b.reshape(5, 1), (5, _CHUNK)),
    )
    parts = [lax.slice(out, (0, _COLOFF[d]), (5, _COLOFF[d] + 2 ** d))
             for d in range(15)]
    parts.append(lax.slice(out, (0, _COLOFF[15]), (5, _COLOFF[15] + _NL)))
    return jnp.concatenate(parts, axis=1).T
